# baseline (speedup 1.0000x reference)
import functools

import jax
import jax.numpy as jnp
from jax import lax
from jax.experimental import pallas as pl
from jax.experimental.pallas import tpu as pltpu
from jax.experimental.pallas import tpu_sc as plsc

_NC = 2
_NS = 16
_NW = _NC * _NS
_CHUNK = 128
_NPAD = 10240
_RPT = _NPAD // _NS
_CW = 16
_ROWS = 512


_NBUF = 2
_NIDX = 4
_GSPLIT = 1


def _sc_segment_sum(d_feat, chunks_per_worker, with_cnt, dtype=jnp.float32):
  cw = chunks_per_worker
  ng = cw // _NIDX
  assert cw % _NIDX == 0 and ng >= 2
  mesh = plsc.VectorSubcoreMesh(core_axis_name="c", subcore_axis_name="s")
  out_type = [jax.ShapeDtypeStruct((_NC * _NPAD, d_feat), dtype)]
  scratch = []
  scratch += [pltpu.VMEM((1, _CHUNK), jnp.int32)] * _NIDX
  scratch += [pltpu.VMEM((1, _CHUNK), jnp.int32)] * _NIDX
  scratch += [pltpu.VMEM((_CHUNK, d_feat), dtype)] * _NBUF
  scratch += [pltpu.SemaphoreType.DMA] * (_NIDX + 2 * _NBUF)
  if with_cnt:
    out_type.append(jax.ShapeDtypeStruct((_NC * _NPAD, _CW), jnp.float32))
    scratch += [
        pltpu.VMEM((_CHUNK, _CW), jnp.float32),
        pltpu.SemaphoreType.DMA,
    ]
  scratch.append(pltpu.VMEM_SHARED((_NPAD, d_feat), dtype))
  scratch.append(pltpu.VMEM_SHARED((_NPAD, d_feat), dtype))
  if with_cnt:
    scratch.append(pltpu.VMEM_SHARED((_NPAD, _CW), jnp.float32))

  def body(src_hbm, dst_hbm, y_hbm, *rest):
    rest = list(rest)
    out_agg = rest.pop(0)
    out_cnt = rest.pop(0) if with_cnt else None
    srcq = [rest.pop(0) for _ in range(_NIDX)]
    dstq = [rest.pop(0) for _ in range(_NIDX)]
    rows = [rest.pop(0) for _ in range(_NBUF)]
    isem = [rest.pop(0) for _ in range(_NIDX)]
    gsem = [rest.pop(0) for _ in range(_NBUF)]
    ssem = [rest.pop(0) for _ in range(_NBUF)]
    if with_cnt:
      ones_v = rest.pop(0)
      csem = rest.pop(0)
    agg_sh = rest.pop(0)
    agg_sh2 = rest.pop(0)
    cnt_sh = rest.pop(0) if with_cnt else None
    aggs = [agg_sh, agg_sh2]
    cid = lax.axis_index("c")
    sid = lax.axis_index("s")
    wid = sid * _NC + cid

    lanes = 32 if dtype == jnp.bfloat16 else 16
    zero = jnp.zeros((lanes,), dtype)
    per_row = d_feat // lanes

    zero16 = jnp.zeros((16,), jnp.float32)

    def zb(j, carry):
      rows[0][j // per_row, pl.ds((j % per_row) * lanes, lanes)] = zero
      return carry

    lax.fori_loop(0, _CHUNK * per_row, zb, 0)
    for j in range(_RPT // _CHUNK):
      pltpu.sync_copy(rows[0],
                      agg_sh.at[pl.ds(sid * _RPT + j * _CHUNK, _CHUNK)])
      pltpu.sync_copy(rows[0],
                      agg_sh2.at[pl.ds(sid * _RPT + j * _CHUNK, _CHUNK)])
    if with_cnt:
      def zc(j, carry):
        ones_v[j, :] = zero16
        return carry

      lax.fori_loop(0, _CHUNK, zc, 0)
      for j in range(_RPT // _CHUNK):
        pltpu.sync_copy(ones_v,
                        cnt_sh.at[pl.ds(sid * _RPT + j * _CHUNK, _CHUNK)])
      one = jnp.ones((16,), jnp.float32)

      def ob(j, carry):
        ones_v[j, :] = one
        return carry

      lax.fori_loop(0, _CHUNK, ob, 0)
    plsc.subcore_barrier()

    base = wid * cw

    def idx_load(q, c):
      pltpu.async_copy(src_hbm.at[pl.ds(base + c, 1)], srcq[q], isem[q])
      pltpu.async_copy(dst_hbm.at[pl.ds(base + c, 1)], dstq[q], isem[q])

    def wait_idx(q):
      pltpu.make_async_copy(src_hbm.at[pl.ds(base, 1)], srcq[q],
                            isem[q]).wait()
      pltpu.make_async_copy(dst_hbm.at[pl.ds(base, 1)], dstq[q],
                            isem[q]).wait()

    gh = _CHUNK // _GSPLIT

    def gather(b, q):
      for p in range(_GSPLIT):
        pltpu.async_copy(y_hbm.at[srcq[q].at[0, pl.ds(p * gh, gh)]],
                         rows[b].at[pl.ds(p * gh, gh)], gsem[b])

    def wait_gather(b):
      for p in range(_GSPLIT):
        pltpu.make_async_copy(y_hbm.at[srcq[0].at[0, pl.ds(0, gh)]],
                              rows[b].at[pl.ds(p * gh, gh)], gsem[b]).wait()

    def scatter(b, q):
      pltpu.async_copy(rows[b], aggs[b].at[dstq[q].at[0]], ssem[b], add=True)
      if with_cnt:
        pltpu.async_copy(ones_v, cnt_sh.at[dstq[q].at[0]], csem, add=True)

    def wait_scatter(b):
      pltpu.make_async_copy(rows[b], aggs[b].at[dstq[0].at[0]],
                            ssem[b]).wait()
      if with_cnt:
        pltpu.make_async_copy(ones_v, cnt_sh.at[dstq[0].at[0]], csem).wait()

    for q in range(_NIDX):
      idx_load(q, q)
    for b in range(_NBUF):
      wait_idx(b)
      gather(b, b)

    def group(g, carry):
      c0 = g * _NIDX
      for j in range(_NIDX):
        b, q = j % _NBUF, j
        wait_gather(b)
        scatter(b, q)
        wait_scatter(b)
        idx_load(q, c0 + j + _NIDX)
        wait_idx((j + _NBUF) % _NIDX)
        gather(b, (j + _NBUF) % _NIDX)
      return carry

    lax.fori_loop(0, ng - 1, group, 0)
    for j in range(_NIDX):
      b, q = j % _NBUF, j
      wait_gather(b)
      scatter(b, q)
      wait_scatter(b)
      if j + _NBUF < _NIDX:
        wait_idx(j + _NBUF)
        gather(b, j + _NBUF)
    plsc.subcore_barrier()

    row0 = sid * _RPT
    vl = 32 if dtype == jnp.bfloat16 else 16
    npr = d_feat // vl
    for j in range(_RPT // _CHUNK):
      sl = pl.ds(row0 + j * _CHUNK, _CHUNK)
      pltpu.sync_copy(agg_sh.at[sl], rows[0])
      pltpu.sync_copy(agg_sh2.at[sl], rows[1])

      def mg(t, carry):
        rr, cc = t // npr, (t % npr) * vl
        rows[0][rr, pl.ds(cc, vl)] = (rows[0][rr, pl.ds(cc, vl)] +
                                      rows[1][rr, pl.ds(cc, vl)])
        return carry

      lax.fori_loop(0, _CHUNK * npr, mg, 0)
      pltpu.sync_copy(rows[0], out_agg.at[pl.ds(cid * _NPAD + row0 + j * _CHUNK,
                                                _CHUNK)])
    if with_cnt:
      pltpu.sync_copy(cnt_sh.at[pl.ds(row0, _RPT)],
                      out_cnt.at[pl.ds(cid * _NPAD + row0, _RPT)])

  return pl.kernel(body, out_type=tuple(out_type), mesh=mesh,
                   scratch_types=tuple(scratch),
                   compiler_params=pltpu.CompilerParams(
                       use_tc_tiling_on_sc=False))


def _row_spec(d):
  return pl.BlockSpec((_ROWS, d), lambda i: (i, 0))


def _full(shape):
  return pl.BlockSpec(shape, lambda i: (0, 0))


def _stage_in(xp, ft_wT, ft_b, ln_g, ln_b, w0_lT):
  def body(x_ref, w_ref, b_ref, g_ref, bb_ref, wl_ref, h_ref, y_ref):
    h = jnp.dot(x_ref[...], w_ref[...], preferred_element_type=jnp.float32)
    h = h + b_ref[...]
    mu = jnp.mean(h, axis=1, keepdims=True)
    var = jnp.mean((h - mu) ** 2, axis=1, keepdims=True)
    h = (h - mu) * lax.rsqrt(var + 1e-5) * g_ref[...] + bb_ref[...]
    h = jnp.where(h >= 0, h, 0.2 * h)
    h_ref[...] = h
    y_ref[...] = jnp.dot(h, wl_ref[...],
                         preferred_element_type=jnp.float32).astype(y_ref.dtype)

  return pl.pallas_call(
      body,
      grid=(_NPAD // _ROWS,),
      in_specs=[_row_spec(128), _full((128, 128)), _full((1, 128)),
                _full((1, 128)), _full((1, 128)), _full((128, 128))],
      out_specs=[_row_spec(128), _row_spec(128)],
      out_shape=[jax.ShapeDtypeStruct((_NPAD, 128), jnp.float32),
                 jax.ShapeDtypeStruct((_NPAD, 128), jnp.bfloat16)],
  )(xp, ft_wT, ft_b, ln_g, ln_b, w0_lT)


def _stage_mid(parts, cnts, h, b_l, bn_g, bn_b, bn_m, bn_v, w_rT, w_nextT,
               d_next, y_dtype):
  def body(p0, p1, c0, c1, h_ref, bl, g, b, m, v, wr, wn, h_out, y_out):
    cnt = c0[:, 0:1] + c1[:, 0:1]
    psum = p0[...].astype(jnp.float32) + p1[...].astype(jnp.float32)
    mean = psum / jnp.maximum(cnt, 1.0)
    z = mean + bl[...] + jnp.dot(h_ref[...], wr[...],
                                 preferred_element_type=jnp.float32)
    z = (z - m[...]) * lax.rsqrt(v[...] + 1e-5) * g[...] + b[...]
    hn = jnp.maximum(z, 0.0)
    h_out[...] = hn
    y_out[...] = jnp.dot(hn, wn[...],
                         preferred_element_type=jnp.float32).astype(y_dtype)

  nb = _NPAD // _ROWS
  p_specs = [pl.BlockSpec((_ROWS, 128), lambda i, k=k: (i + k * nb, 0))
             for k in range(2)]
  c_spec0 = pl.BlockSpec((_ROWS, _CW), lambda i: (i, 0))
  c_spec1 = pl.BlockSpec((_ROWS, _CW), lambda i: (i + nb, 0))
  return pl.pallas_call(
      body,
      grid=(nb,),
      in_specs=p_specs + [c_spec0, c_spec1, _row_spec(128),
                          _full((1, 128)), _full((1, 128)), _full((1, 128)),
                          _full((1, 128)), _full((1, 128)), _full((128, 128)),
                          _full((128, d_next))],
      out_specs=[_row_spec(128), _row_spec(d_next)],
      out_shape=[jax.ShapeDtypeStruct((_NPAD, 128), jnp.float32),
                 jax.ShapeDtypeStruct((_NPAD, d_next), y_dtype)],
  )(parts, parts, cnts, cnts, h, b_l, bn_g, bn_b, bn_m, bn_v, w_rT, w_nextT)


def _stage_out(parts, cnts, h, b2, w_rT):
  def body(p0, p1, c0, c1, h_ref, bl, wr, o_ref):
    cnt = c0[:, 0:1] + c1[:, 0:1]
    mean = (p0[...] + p1[...]) / jnp.maximum(cnt, 1.0)
    o_ref[...] = mean + bl[...] + jnp.dot(h_ref[...], wr[...],
                                          preferred_element_type=jnp.float32)

  nb = _NPAD // _ROWS
  p_specs = [pl.BlockSpec((_ROWS, _CW), lambda i, k=k: (i + k * nb, 0))
             for k in range(2)]
  c_spec0 = pl.BlockSpec((_ROWS, _CW), lambda i: (i, 0))
  c_spec1 = pl.BlockSpec((_ROWS, _CW), lambda i: (i + nb, 0))
  return pl.pallas_call(
      body,
      grid=(nb,),
      in_specs=p_specs + [c_spec0, c_spec1, _row_spec(128),
                          _full((1, _CW)), _full((128, _CW))],
      out_specs=_row_spec(_CW),
      out_shape=jax.ShapeDtypeStruct((_NPAD, _CW), jnp.float32),
  )(parts, parts, cnts, cnts, h, b2, w_rT)


def kernel(x, edge_index, ft_w, ft_b, ln_g, ln_b, w0_l, b0_l, w0_r, bn0_g,
           bn0_b, bn0_m, bn0_v, w1_l, b1_l, w1_r, bn1_g, bn1_b, bn1_m, bn1_v,
           w2_l, b2_l, w2_r):
  n, _ = x.shape
  src = edge_index[0]
  dst = edge_index[1]
  e = src.shape[0]
  cpw = -(-e // (_NW * _CHUNK))
  cpw = _NIDX * (-(-cpw // _NIDX))
  e_pad = _NW * cpw * _CHUNK
  pad_idx = jnp.full((e_pad - e,), n, jnp.int32)
  src_p = jnp.concatenate([src, pad_idx]).reshape(-1, _CHUNK)
  dst_p = jnp.concatenate([dst, pad_idx]).reshape(-1, _CHUNK)
  xp = jnp.pad(x, ((0, _NPAD - n), (0, 0)))

  r = lambda v: v.reshape(1, -1)
  w2_lp = jnp.zeros((_CW, 128), jnp.float32).at[:2].set(w2_l)
  w2_rp = jnp.zeros((_CW, 128), jnp.float32).at[:2].set(w2_r)
  b2_p = jnp.zeros((_CW,), jnp.float32).at[:2].set(b2_l)

  h0, y0 = _stage_in(xp, ft_w.T, r(ft_b), r(ln_g), r(ln_b), w0_l.T)
  agg0, cnt = _sc_segment_sum(128, cpw, True, jnp.bfloat16)(src_p, dst_p, y0)
  h1, y1 = _stage_mid(agg0, cnt, h0, r(b0_l), r(bn0_g), r(bn0_b), r(bn0_m),
                      r(bn0_v), w0_r.T, w1_l.T, 128, jnp.bfloat16)
  (agg1,) = _sc_segment_sum(128, cpw, False, jnp.bfloat16)(src_p, dst_p, y1)
  h2, y2 = _stage_mid(agg1, cnt, h1, r(b1_l), r(bn1_g), r(bn1_b), r(bn1_m),
                      r(bn1_v), w1_r.T, w2_lp.T, _CW, jnp.float32)
  (agg2,) = _sc_segment_sum(_CW, cpw, False)(src_p, dst_p, y2)
  outp = _stage_out(agg2, cnt, h2, r(b2_p), w2_rp.T)
  return outp[:n, :2]


# speedup vs baseline: 1.0765x; 1.0765x over previous
import functools

import jax
import jax.numpy as jnp
from jax import lax
from jax.experimental import pallas as pl
from jax.experimental.pallas import tpu as pltpu
from jax.experimental.pallas import tpu_sc as plsc

_NC = 2
_NS = 16
_NW = _NC * _NS
_CHUNK = 128
_NPAD = 10240
_RPT = _NPAD // _NS
_CW = 16
_ROWS = 512


_NBUF = 2
_NIDX = 4
_GSPLIT = 1


def _sc_segment_sum(d_feat, chunks_per_worker, with_cnt, dtype=jnp.float32):
  cw = chunks_per_worker
  ng = cw // _NIDX
  assert cw % _NIDX == 0 and ng >= 2
  mesh = plsc.VectorSubcoreMesh(core_axis_name="c", subcore_axis_name="s")
  out_type = [jax.ShapeDtypeStruct((2 * _NC * _NPAD, d_feat), dtype)]
  scratch = []
  scratch += [pltpu.VMEM((1, _CHUNK), jnp.int32)] * _NIDX
  scratch += [pltpu.VMEM((1, _CHUNK), jnp.int32)] * _NIDX
  scratch += [pltpu.VMEM((_CHUNK, d_feat), dtype)] * _NBUF
  scratch += [pltpu.SemaphoreType.DMA] * (_NIDX + 2 * _NBUF)
  if with_cnt:
    out_type.append(jax.ShapeDtypeStruct((_NC * _NPAD, _CW), jnp.float32))
    scratch += [
        pltpu.VMEM((_CHUNK, _CW), jnp.float32),
        pltpu.SemaphoreType.DMA,
    ]
  scratch.append(pltpu.VMEM_SHARED((_NPAD, d_feat), dtype))
  scratch.append(pltpu.VMEM_SHARED((_NPAD, d_feat), dtype))
  if with_cnt:
    scratch.append(pltpu.VMEM_SHARED((_NPAD, _CW), jnp.float32))

  def body(src_hbm, dst_hbm, y_hbm, *rest):
    rest = list(rest)
    out_agg = rest.pop(0)
    out_cnt = rest.pop(0) if with_cnt else None
    srcq = [rest.pop(0) for _ in range(_NIDX)]
    dstq = [rest.pop(0) for _ in range(_NIDX)]
    rows = [rest.pop(0) for _ in range(_NBUF)]
    isem = [rest.pop(0) for _ in range(_NIDX)]
    gsem = [rest.pop(0) for _ in range(_NBUF)]
    ssem = [rest.pop(0) for _ in range(_NBUF)]
    if with_cnt:
      ones_v = rest.pop(0)
      csem = rest.pop(0)
    agg_sh = rest.pop(0)
    agg_sh2 = rest.pop(0)
    cnt_sh = rest.pop(0) if with_cnt else None
    aggs = [agg_sh, agg_sh2]
    cid = lax.axis_index("c")
    sid = lax.axis_index("s")
    wid = sid * _NC + cid

    lanes = 32 if dtype == jnp.bfloat16 else 16
    zero = jnp.zeros((lanes,), dtype)
    per_row = d_feat // lanes

    zero16 = jnp.zeros((16,), jnp.float32)

    def zb(j, carry):
      rows[0][j // per_row, pl.ds((j % per_row) * lanes, lanes)] = zero
      return carry

    lax.fori_loop(0, _CHUNK * per_row, zb, 0)
    for j in range(_RPT // _CHUNK):
      pltpu.sync_copy(rows[0],
                      agg_sh.at[pl.ds(sid * _RPT + j * _CHUNK, _CHUNK)])
      pltpu.sync_copy(rows[0],
                      agg_sh2.at[pl.ds(sid * _RPT + j * _CHUNK, _CHUNK)])
    if with_cnt:
      def zc(j, carry):
        ones_v[j, :] = zero16
        return carry

      lax.fori_loop(0, _CHUNK, zc, 0)
      for j in range(_RPT // _CHUNK):
        pltpu.sync_copy(ones_v,
                        cnt_sh.at[pl.ds(sid * _RPT + j * _CHUNK, _CHUNK)])
      one = jnp.ones((16,), jnp.float32)

      def ob(j, carry):
        ones_v[j, :] = one
        return carry

      lax.fori_loop(0, _CHUNK, ob, 0)
    plsc.subcore_barrier()

    base = wid * cw

    def idx_load(q, c):
      pltpu.async_copy(src_hbm.at[pl.ds(base + c, 1)], srcq[q], isem[q])
      pltpu.async_copy(dst_hbm.at[pl.ds(base + c, 1)], dstq[q], isem[q])

    def wait_idx(q):
      pltpu.make_async_copy(src_hbm.at[pl.ds(base, 1)], srcq[q],
                            isem[q]).wait()
      pltpu.make_async_copy(dst_hbm.at[pl.ds(base, 1)], dstq[q],
                            isem[q]).wait()

    gh = _CHUNK // _GSPLIT

    def gather(b, q):
      for p in range(_GSPLIT):
        pltpu.async_copy(y_hbm.at[srcq[q].at[0, pl.ds(p * gh, gh)]],
                         rows[b].at[pl.ds(p * gh, gh)], gsem[b])

    def wait_gather(b):
      for p in range(_GSPLIT):
        pltpu.make_async_copy(y_hbm.at[srcq[0].at[0, pl.ds(0, gh)]],
                              rows[b].at[pl.ds(p * gh, gh)], gsem[b]).wait()

    def scatter(b, q):
      pltpu.async_copy(rows[b], aggs[b].at[dstq[q].at[0]], ssem[b], add=True)
      if with_cnt:
        pltpu.async_copy(ones_v, cnt_sh.at[dstq[q].at[0]], csem, add=True)

    def wait_scatter(b):
      pltpu.make_async_copy(rows[b], aggs[b].at[dstq[0].at[0]],
                            ssem[b]).wait()
      if with_cnt:
        pltpu.make_async_copy(ones_v, cnt_sh.at[dstq[0].at[0]], csem).wait()

    for q in range(_NIDX):
      idx_load(q, q)
    for b in range(_NBUF):
      wait_idx(b)
      gather(b, b)

    def group(g, carry):
      c0 = g * _NIDX
      for j in range(_NIDX):
        b, q = j % _NBUF, j
        wait_gather(b)
        scatter(b, q)
        wait_scatter(b)
        idx_load(q, c0 + j + _NIDX)
        wait_idx((j + _NBUF) % _NIDX)
        gather(b, (j + _NBUF) % _NIDX)
      return carry

    lax.fori_loop(0, ng - 1, group, 0)
    for j in range(_NIDX):
      b, q = j % _NBUF, j
      wait_gather(b)
      scatter(b, q)
      wait_scatter(b)
      if j + _NBUF < _NIDX:
        wait_idx(j + _NBUF)
        gather(b, j + _NBUF)
    plsc.subcore_barrier()

    row0 = sid * _RPT
    pltpu.sync_copy(agg_sh.at[pl.ds(row0, _RPT)],
                    out_agg.at[pl.ds(cid * _NPAD + row0, _RPT)])
    pltpu.sync_copy(agg_sh2.at[pl.ds(row0, _RPT)],
                    out_agg.at[pl.ds((_NC + cid) * _NPAD + row0, _RPT)])
    if with_cnt:
      pltpu.sync_copy(cnt_sh.at[pl.ds(row0, _RPT)],
                      out_cnt.at[pl.ds(cid * _NPAD + row0, _RPT)])

  return pl.kernel(body, out_type=tuple(out_type), mesh=mesh,
                   scratch_types=tuple(scratch),
                   compiler_params=pltpu.CompilerParams(
                       use_tc_tiling_on_sc=False))


def _row_spec(d):
  return pl.BlockSpec((_ROWS, d), lambda i: (i, 0))


def _full(shape):
  return pl.BlockSpec(shape, lambda i: (0, 0))


def _stage_in(xp, ft_wT, ft_b, ln_g, ln_b, w0_lT):
  def body(x_ref, w_ref, b_ref, g_ref, bb_ref, wl_ref, h_ref, y_ref):
    h = jnp.dot(x_ref[...], w_ref[...], preferred_element_type=jnp.float32)
    h = h + b_ref[...]
    mu = jnp.mean(h, axis=1, keepdims=True)
    var = jnp.mean((h - mu) ** 2, axis=1, keepdims=True)
    h = (h - mu) * lax.rsqrt(var + 1e-5) * g_ref[...] + bb_ref[...]
    h = jnp.where(h >= 0, h, 0.2 * h)
    h_ref[...] = h
    y_ref[...] = jnp.dot(h, wl_ref[...],
                         preferred_element_type=jnp.float32).astype(y_ref.dtype)

  return pl.pallas_call(
      body,
      grid=(_NPAD // _ROWS,),
      in_specs=[_row_spec(128), _full((128, 128)), _full((1, 128)),
                _full((1, 128)), _full((1, 128)), _full((128, 128))],
      out_specs=[_row_spec(128), _row_spec(128)],
      out_shape=[jax.ShapeDtypeStruct((_NPAD, 128), jnp.float32),
                 jax.ShapeDtypeStruct((_NPAD, 128), jnp.bfloat16)],
  )(xp, ft_wT, ft_b, ln_g, ln_b, w0_lT)


def _stage_mid(parts, cnts, h, b_l, bn_g, bn_b, bn_m, bn_v, w_rT, w_nextT,
               d_next, y_dtype):
  def body(p0, p1, p2, p3, c0, c1, h_ref, bl, g, b, m, v, wr, wn, h_out,
           y_out):
    cnt = c0[:, 0:1] + c1[:, 0:1]
    psum = (p0[...].astype(jnp.float32) + p1[...].astype(jnp.float32) +
            p2[...].astype(jnp.float32) + p3[...].astype(jnp.float32))
    mean = psum / jnp.maximum(cnt, 1.0)
    z = mean + bl[...] + jnp.dot(h_ref[...], wr[...],
                                 preferred_element_type=jnp.float32)
    z = (z - m[...]) * lax.rsqrt(v[...] + 1e-5) * g[...] + b[...]
    hn = jnp.maximum(z, 0.0)
    h_out[...] = hn
    y_out[...] = jnp.dot(hn, wn[...],
                         preferred_element_type=jnp.float32).astype(y_dtype)

  nb = _NPAD // _ROWS
  p_specs = [pl.BlockSpec((_ROWS, 128), lambda i, k=k: (i + k * nb, 0))
             for k in range(4)]
  c_spec0 = pl.BlockSpec((_ROWS, _CW), lambda i: (i, 0))
  c_spec1 = pl.BlockSpec((_ROWS, _CW), lambda i: (i + nb, 0))
  return pl.pallas_call(
      body,
      grid=(nb,),
      in_specs=p_specs + [c_spec0, c_spec1, _row_spec(128),
                          _full((1, 128)), _full((1, 128)), _full((1, 128)),
                          _full((1, 128)), _full((1, 128)), _full((128, 128)),
                          _full((128, d_next))],
      out_specs=[_row_spec(128), _row_spec(d_next)],
      out_shape=[jax.ShapeDtypeStruct((_NPAD, 128), jnp.float32),
                 jax.ShapeDtypeStruct((_NPAD, d_next), y_dtype)],
  )(parts, parts, parts, parts, cnts, cnts, h, b_l, bn_g, bn_b, bn_m, bn_v,
    w_rT, w_nextT)


def _stage_out(parts, cnts, h, b2, w_rT):
  def body(p0, p1, p2, p3, c0, c1, h_ref, bl, wr, o_ref):
    cnt = c0[:, 0:1] + c1[:, 0:1]
    mean = (p0[...] + p1[...] + p2[...] + p3[...]) / jnp.maximum(cnt, 1.0)
    o_ref[...] = mean + bl[...] + jnp.dot(h_ref[...], wr[...],
                                          preferred_element_type=jnp.float32)

  nb = _NPAD // _ROWS
  p_specs = [pl.BlockSpec((_ROWS, _CW), lambda i, k=k: (i + k * nb, 0))
             for k in range(4)]
  c_spec0 = pl.BlockSpec((_ROWS, _CW), lambda i: (i, 0))
  c_spec1 = pl.BlockSpec((_ROWS, _CW), lambda i: (i + nb, 0))
  return pl.pallas_call(
      body,
      grid=(nb,),
      in_specs=p_specs + [c_spec0, c_spec1, _row_spec(128),
                          _full((1, _CW)), _full((128, _CW))],
      out_specs=_row_spec(_CW),
      out_shape=jax.ShapeDtypeStruct((_NPAD, _CW), jnp.float32),
  )(parts, parts, parts, parts, cnts, cnts, h, b2, w_rT)


def kernel(x, edge_index, ft_w, ft_b, ln_g, ln_b, w0_l, b0_l, w0_r, bn0_g,
           bn0_b, bn0_m, bn0_v, w1_l, b1_l, w1_r, bn1_g, bn1_b, bn1_m, bn1_v,
           w2_l, b2_l, w2_r):
  n, _ = x.shape
  src = edge_index[0]
  dst = edge_index[1]
  e = src.shape[0]
  cpw = -(-e // (_NW * _CHUNK))
  cpw = _NIDX * (-(-cpw // _NIDX))
  e_pad = _NW * cpw * _CHUNK
  pad_idx = jnp.full((e_pad - e,), n, jnp.int32)
  src_p = jnp.concatenate([src, pad_idx]).reshape(-1, _CHUNK)
  dst_p = jnp.concatenate([dst, pad_idx]).reshape(-1, _CHUNK)
  xp = jnp.pad(x, ((0, _NPAD - n), (0, 0)))

  r = lambda v: v.reshape(1, -1)
  w2_lp = jnp.zeros((_CW, 128), jnp.float32).at[:2].set(w2_l)
  w2_rp = jnp.zeros((_CW, 128), jnp.float32).at[:2].set(w2_r)
  b2_p = jnp.zeros((_CW,), jnp.float32).at[:2].set(b2_l)

  h0, y0 = _stage_in(xp, ft_w.T, r(ft_b), r(ln_g), r(ln_b), w0_l.T)
  agg0, cnt = _sc_segment_sum(128, cpw, True, jnp.bfloat16)(src_p, dst_p, y0)
  h1, y1 = _stage_mid(agg0, cnt, h0, r(b0_l), r(bn0_g), r(bn0_b), r(bn0_m),
                      r(bn0_v), w0_r.T, w1_l.T, 128, jnp.bfloat16)
  (agg1,) = _sc_segment_sum(128, cpw, False, jnp.bfloat16)(src_p, dst_p, y1)
  h2, y2 = _stage_mid(agg1, cnt, h1, r(b1_l), r(bn1_g), r(bn1_b), r(bn1_m),
                      r(bn1_v), w1_r.T, w2_lp.T, _CW, jnp.float32)
  (agg2,) = _sc_segment_sum(_CW, cpw, False)(src_p, dst_p, y2)
  outp = _stage_out(agg2, cnt, h2, r(b2_p), w2_rp.T)
  return outp[:n, :2]
